# baseline (speedup 1.0000x reference)
import functools

import jax
import jax.numpy as jnp
from jax import lax
from jax.experimental import pallas as pl
from jax.experimental.pallas import tpu as pltpu
from jax.experimental.pallas import tpu_sc as plsc

_N = 16384
_D = 2048
_E = 64
_R = 2048

_CHUNKS = 4
_CH = _N // _CHUNKS

_NW = 32
_C = _CH // _NW
_L = 16
_G = _C // _L


def _logits_block(x_ref, w_ref, b_ref, out_ref):
    out_ref[...] = lax.dot_general(
        w_ref[...], x_ref[...],
        dimension_numbers=(((0,), (1,)), ((), ())),
        preferred_element_type=jnp.float32,
    ) + b_ref[...]


def _logits_chunk(x, w, b, chunk):
    base_blk = chunk * (_CH // _R)
    return pl.pallas_call(
        _logits_block,
        grid=(_CH // _R,),
        in_specs=[
            pl.BlockSpec((_R, _D), lambda i: (base_blk + i, 0)),
            pl.BlockSpec((_D, _E), lambda i: (0, 0)),
            pl.BlockSpec((_E, 1), lambda i: (0, 0)),
        ],
        out_specs=pl.BlockSpec((_E, _R), lambda i: (0, i)),
        out_shape=jax.ShapeDtypeStruct((_E, _CH), jnp.float32),
        compiler_params=pltpu.CompilerParams(
            dimension_semantics=("arbitrary",),
        ),
    )(x, w, b.reshape(_E, 1))


@functools.partial(
    pl.kernel,
    out_type=[
        jax.ShapeDtypeStruct((2, _CH), jnp.int32),
        jax.ShapeDtypeStruct((2, _CH), jnp.float32),
    ],
    mesh=plsc.VectorSubcoreMesh(core_axis_name="c", subcore_axis_name="s"),
    scratch_types=[
        pltpu.VMEM((_E, _C), jnp.float32),
        pltpu.VMEM((_C,), jnp.int32),
        pltpu.VMEM((_C,), jnp.int32),
        pltpu.VMEM((_C,), jnp.float32),
        pltpu.VMEM((_C,), jnp.float32),
    ],
)
def _sc_top2(logits_hbm, idx_hbm, probs_hbm, lbuf, i1b, i2b, p1b, p2b):
    wid = lax.axis_index("s") * 2 + lax.axis_index("c")
    base = wid * _C
    pltpu.sync_copy(logits_hbm.at[:, pl.ds(base, _C)], lbuf)

    def group(g, carry):
        sl = pl.ds(g * _L, _L)
        m1 = lbuf[0, sl]
        i1 = jnp.zeros((_L,), jnp.int32)
        m2 = jnp.full((_L,), -jnp.inf, jnp.float32)
        i2 = jnp.zeros((_L,), jnp.int32)
        for e in range(1, _E):
            v = lbuf[e, sl]
            ei = jnp.full((_L,), e, jnp.int32)
            new1 = v > m1
            new2 = v > m2
            m2 = jnp.where(new1, m1, jnp.where(new2, v, m2))
            i2 = jnp.where(new1, i1, jnp.where(new2, ei, i2))
            m1 = jnp.where(new1, v, m1)
            i1 = jnp.where(new1, ei, i1)
        ex = jnp.exp(m2 - m1)
        den = 1.0 + ex
        i1b[sl] = i1
        i2b[sl] = i2
        p1b[sl] = 1.0 / den
        p2b[sl] = ex / den
        return carry

    lax.fori_loop(0, _G, group, 0)
    pltpu.sync_copy(i1b, idx_hbm.at[0, pl.ds(base, _C)])
    pltpu.sync_copy(i2b, idx_hbm.at[1, pl.ds(base, _C)])
    pltpu.sync_copy(p1b, probs_hbm.at[0, pl.ds(base, _C)])
    pltpu.sync_copy(p2b, probs_hbm.at[1, pl.ds(base, _C)])


def kernel(x, W_gate, b_gate):
    idx_parts, prob_parts = [], []
    for c in range(_CHUNKS):
        logits_t = _logits_chunk(x, W_gate, b_gate, c)
        idx_t, probs_t = _sc_top2(logits_t)
        idx_parts.append(idx_t)
        prob_parts.append(probs_t)
    idx = jnp.concatenate(idx_parts, axis=1).T
    probs = jnp.concatenate(prob_parts, axis=1).T
    return (idx, probs)


# speedup vs baseline: 1.2121x; 1.2121x over previous
import functools

import jax
import jax.numpy as jnp
from jax import lax
from jax.experimental import pallas as pl
from jax.experimental.pallas import tpu as pltpu
from jax.experimental.pallas import tpu_sc as plsc

_N = 16384
_D = 2048
_E = 64
_R = 2048

_NW = 32
_C = _N // _NW
_L = 16
_G = _C // _L

def _keys_block(x_ref, w_ref, b_ref, out_ref):
    logits = lax.dot_general(
        w_ref[...], x_ref[...],
        dimension_numbers=(((0,), (1,)), ((), ())),
        preferred_element_type=jnp.float32,
    ) + b_ref[...]
    bits = lax.bitcast_convert_type(logits, jnp.int32)
    s = bits ^ (lax.shift_right_arithmetic(bits, 31) & 0x7FFFFFFF)
    erow = lax.broadcasted_iota(jnp.int32, logits.shape, 0)
    out_ref[...] = (s & ~63) | (63 - erow)


def _keys_T(x, w, b):
    return pl.pallas_call(
        _keys_block,
        grid=(_N // _R,),
        in_specs=[
            pl.BlockSpec((_R, _D), lambda i: (i, 0)),
            pl.BlockSpec((_D, _E), lambda i: (0, 0)),
            pl.BlockSpec((_E, 1), lambda i: (0, 0)),
        ],
        out_specs=pl.BlockSpec((_E, _R), lambda i: (0, i)),
        out_shape=jax.ShapeDtypeStruct((_E, _N), jnp.int32),
        compiler_params=pltpu.CompilerParams(
            dimension_semantics=("arbitrary",),
        ),
    )(x, w, b.reshape(_E, 1))


def _key_to_logit(key):
    s = key & ~63
    bits = s ^ (lax.shift_right_arithmetic(s, 31) & 0x7FFFFFFF)
    return lax.bitcast_convert_type(bits, jnp.float32)


@functools.partial(
    pl.kernel,
    out_type=[
        jax.ShapeDtypeStruct((2, _N), jnp.int32),
        jax.ShapeDtypeStruct((2, _N), jnp.float32),
    ],
    mesh=plsc.VectorSubcoreMesh(core_axis_name="c", subcore_axis_name="s"),
    scratch_types=[
        pltpu.VMEM((_E, _C), jnp.int32),
        pltpu.VMEM((_C,), jnp.int32),
        pltpu.VMEM((_C,), jnp.int32),
        pltpu.VMEM((_C,), jnp.float32),
        pltpu.VMEM((_C,), jnp.float32),
    ],
)
def _sc_top2(keys_hbm, idx_hbm, probs_hbm, kbuf, i1b, i2b, p1b, p2b):
    wid = lax.axis_index("s") * 2 + lax.axis_index("c")
    base = wid * _C
    pltpu.sync_copy(keys_hbm.at[:, pl.ds(base, _C)], kbuf)

    def group(g, carry):
        sl = pl.ds(g * _L, _L)
        m1 = kbuf[0, sl]
        m2 = jnp.full((_L,), jnp.iinfo(jnp.int32).min, jnp.int32)
        for e in range(1, _E):
            k = kbuf[e, sl]
            m2 = jnp.maximum(m2, jnp.minimum(m1, k))
            m1 = jnp.maximum(m1, k)
        i1b[sl] = 63 - (m1 & 63)
        i2b[sl] = 63 - (m2 & 63)
        ex = jnp.exp(_key_to_logit(m2) - _key_to_logit(m1))
        den = 1.0 + ex
        p1b[sl] = 1.0 / den
        p2b[sl] = ex / den
        return carry

    lax.fori_loop(0, _G, group, 0)
    pltpu.sync_copy(i1b, idx_hbm.at[0, pl.ds(base, _C)])
    pltpu.sync_copy(i2b, idx_hbm.at[1, pl.ds(base, _C)])
    pltpu.sync_copy(p1b, probs_hbm.at[0, pl.ds(base, _C)])
    pltpu.sync_copy(p2b, probs_hbm.at[1, pl.ds(base, _C)])


def kernel(x, W_gate, b_gate):
    keys_t = _keys_T(x, W_gate, b_gate)
    idx_t, probs_t = _sc_top2(keys_t)
    return (idx_t.T, probs_t.T)
